# baseline (speedup 1.0000x reference)
import jax
import jax.numpy as jnp
from jax.experimental import pallas as pl
from jax.experimental.pallas import tpu as pltpu

N, D = 16384, 64
BC = 8192
NB = N // BC
NBUF = 2


def _body(u_hbm, i_hbm, o_hbm, *rest):
    ubufs = rest[0:NBUF]
    ibufs = rest[NBUF:2 * NBUF]
    o_v = rest[2 * NBUF]
    sems = rest[2 * NBUF + 1:2 * NBUF + 1 + NBUF]
    osem = rest[2 * NBUF + 1 + NBUF]

    def start(k):
        b = k % NBUF
        cu = pltpu.make_async_copy(
            u_hbm.at[:, pl.ds(k * BC, BC)], ubufs[b], sems[b])
        ci = pltpu.make_async_copy(
            i_hbm.at[:, pl.ds(k * BC, BC)], ibufs[b], sems[b])
        cu.start()
        ci.start()
        return cu, ci

    pend = [start(k) for k in range(NBUF - 1)]
    for k in range(NB):
        if k + NBUF - 1 < NB:
            pend.append(start(k + NBUF - 1))
        cu, ci = pend.pop(0)
        cu.wait()
        ci.wait()
        b = k % NBUF
        o_v[pl.ds(k * BC, BC)] = jnp.sum(ubufs[b][...] * ibufs[b][...], axis=0)
    out_cp = pltpu.make_async_copy(o_v, o_hbm, osem)
    out_cp.start()
    out_cp.wait()


def kernel(gu, gi):
    return pl.pallas_call(
        _body,
        in_specs=[
            pl.BlockSpec(memory_space=pltpu.HBM),
            pl.BlockSpec(memory_space=pltpu.HBM),
        ],
        out_specs=pl.BlockSpec(memory_space=pltpu.HBM),
        out_shape=jax.ShapeDtypeStruct((N,), jnp.float32),
        scratch_shapes=(
            [pltpu.VMEM((D, BC), jnp.float32) for _ in range(2 * NBUF)]
            + [pltpu.VMEM((N,), jnp.float32)]
            + [pltpu.SemaphoreType.DMA for _ in range(NBUF + 1)]
        ),
    )(pltpu.with_memory_space_constraint(gu.T, pltpu.HBM),
      pltpu.with_memory_space_constraint(gi.T, pltpu.HBM))


# speedup vs baseline: 1.0411x; 1.0411x over previous
import jax
import jax.numpy as jnp
from jax.experimental import pallas as pl
from jax.experimental.pallas import tpu as pltpu

N, D = 16384, 64
BC = 4096
NB = N // BC
NBUF = 4


def _body(u_hbm, i_hbm, o_hbm, *rest):
    ubufs = rest[0:NBUF]
    ibufs = rest[NBUF:2 * NBUF]
    o_v = rest[2 * NBUF]
    sems = rest[2 * NBUF + 1:2 * NBUF + 1 + NBUF]
    osem = rest[2 * NBUF + 1 + NBUF]

    def start(k):
        b = k % NBUF
        cu = pltpu.make_async_copy(
            u_hbm.at[:, pl.ds(k * BC, BC)], ubufs[b], sems[b])
        ci = pltpu.make_async_copy(
            i_hbm.at[:, pl.ds(k * BC, BC)], ibufs[b], sems[b])
        cu.start()
        ci.start()
        return cu, ci

    pend = [start(k) for k in range(NBUF - 1)]
    for k in range(NB):
        if k + NBUF - 1 < NB:
            pend.append(start(k + NBUF - 1))
        cu, ci = pend.pop(0)
        cu.wait()
        ci.wait()
        b = k % NBUF
        o_v[pl.ds(k * BC, BC)] = jnp.sum(ubufs[b][...] * ibufs[b][...], axis=0)
    out_cp = pltpu.make_async_copy(o_v, o_hbm, osem)
    out_cp.start()
    out_cp.wait()


def kernel(gu, gi):
    return pl.pallas_call(
        _body,
        in_specs=[
            pl.BlockSpec(memory_space=pltpu.HBM),
            pl.BlockSpec(memory_space=pltpu.HBM),
        ],
        out_specs=pl.BlockSpec(memory_space=pltpu.HBM),
        out_shape=jax.ShapeDtypeStruct((N,), jnp.float32),
        scratch_shapes=(
            [pltpu.VMEM((D, BC), jnp.float32) for _ in range(2 * NBUF)]
            + [pltpu.VMEM((N,), jnp.float32)]
            + [pltpu.SemaphoreType.DMA for _ in range(NBUF + 1)]
        ),
    )(pltpu.with_memory_space_constraint(gu.T, pltpu.HBM),
      pltpu.with_memory_space_constraint(gi.T, pltpu.HBM))
